# Initial kernel scaffold; baseline (speedup 1.0000x reference)
import jax
import jax.numpy as jnp
from jax.experimental import pallas as pl


def kernel(x, W_enc, b_enc, W1, b1, W2, b2, W3, b3, ln_w, ln_b):
    raise NotImplementedError("write your pallas kernel here")


# speedup vs baseline: 1.6318x; 1.6318x over previous
import jax
import jax.numpy as jnp
from jax.experimental import pallas as pl


def _gelu_exact(x):
    return 0.5 * x * (1.0 + jax.lax.erf(x * 0.7071067811865476))


def _body(x_ref, wenc_ref, benc_ref, w1_ref, b1_ref, w2_ref, b2_ref,
          w3_ref, b3_ref, lnw_ref, lnb_ref, out_ref):
    hi = jax.lax.Precision.HIGHEST
    x = x_ref[...]
    logits = jnp.dot(x, wenc_ref[...],
                     preferred_element_type=jnp.float32) + benc_ref[...]
    bits = jnp.sign(logits)
    h = jnp.dot(bits, w1_ref[...], precision=hi,
                preferred_element_type=jnp.float32) + b1_ref[...]
    h = _gelu_exact(h)
    h = jnp.dot(h, w2_ref[...], precision=hi,
                preferred_element_type=jnp.float32) + b2_ref[...]
    h = _gelu_exact(h)
    h = jnp.dot(h, w3_ref[...], precision=hi,
                preferred_element_type=jnp.float32) + b3_ref[...]
    mean = jnp.mean(h, axis=-1, keepdims=True)
    var = jnp.mean((h - mean) * (h - mean), axis=-1, keepdims=True)
    out_ref[...] = ((h - mean) * jax.lax.rsqrt(var + 1e-5)
                    * lnw_ref[...] + lnb_ref[...])


def kernel(x, W_enc, b_enc, W1, b1, W2, b2, W3, b3, ln_w, ln_b,
           block_m: int = 512, interpret: bool = False):
    B, T, D = x.shape
    K = W_enc.shape[0]
    H = W1.shape[0]
    M = B * T
    xf = x.reshape(M, D)

    rep = lambda i: (0, 0)
    out = pl.pallas_call(
        _body,
        grid=(M // block_m,),
        in_specs=[
            pl.BlockSpec((block_m, D), lambda i: (i, 0)),
            pl.BlockSpec((D, K), rep),
            pl.BlockSpec((1, K), rep),
            pl.BlockSpec((K, H), rep),
            pl.BlockSpec((1, H), rep),
            pl.BlockSpec((H, H), rep),
            pl.BlockSpec((1, H), rep),
            pl.BlockSpec((H, D), rep),
            pl.BlockSpec((1, D), rep),
            pl.BlockSpec((1, D), rep),
            pl.BlockSpec((1, D), rep),
        ],
        out_specs=pl.BlockSpec((block_m, D), lambda i: (i, 0)),
        out_shape=jax.ShapeDtypeStruct((M, D), jnp.float32),
        interpret=interpret,
    )(xf, W_enc.T, b_enc.reshape(1, K), W1.T, b1.reshape(1, H),
      W2.T, b2.reshape(1, H), W3.T, b3.reshape(1, D),
      ln_w.reshape(1, D), ln_b.reshape(1, D))
    return out.reshape(B, T, D)


# speedup vs baseline: 4.3888x; 2.6895x over previous
import jax
import jax.numpy as jnp
from jax.experimental import pallas as pl


def _gelu_exact(x):
    return 0.5 * x * (1.0 + jax.lax.erf(x * 0.7071067811865476))


def _body(x_ref, wenc_ref, benc_ref, w1_ref, b1_ref, w2_ref, b2_ref,
          w3_ref, b3_ref, lnw_ref, lnb_ref, out_ref):
    x = x_ref[...]
    logits = jnp.dot(x, wenc_ref[...],
                     preferred_element_type=jnp.float32) + benc_ref[...]
    bits = jnp.sign(logits)
    h = jnp.dot(bits, w1_ref[...],
                preferred_element_type=jnp.float32) + b1_ref[...]
    h = _gelu_exact(h)
    h = jnp.dot(h, w2_ref[...],
                preferred_element_type=jnp.float32) + b2_ref[...]
    h = _gelu_exact(h)
    h = jnp.dot(h, w3_ref[...],
                preferred_element_type=jnp.float32) + b3_ref[...]
    mean = jnp.mean(h, axis=-1, keepdims=True)
    var = jnp.mean((h - mean) * (h - mean), axis=-1, keepdims=True)
    out_ref[...] = ((h - mean) * jax.lax.rsqrt(var + 1e-5)
                    * lnw_ref[...] + lnb_ref[...])


def kernel(x, W_enc, b_enc, W1, b1, W2, b2, W3, b3, ln_w, ln_b,
           block_m: int = 512, interpret: bool = False):
    B, T, D = x.shape
    K = W_enc.shape[0]
    H = W1.shape[0]
    M = B * T
    xf = x.reshape(M, D)

    rep = lambda i: (0, 0)
    out = pl.pallas_call(
        _body,
        grid=(M // block_m,),
        in_specs=[
            pl.BlockSpec((block_m, D), lambda i: (i, 0)),
            pl.BlockSpec((D, K), rep),
            pl.BlockSpec((1, K), rep),
            pl.BlockSpec((K, H), rep),
            pl.BlockSpec((1, H), rep),
            pl.BlockSpec((H, H), rep),
            pl.BlockSpec((1, H), rep),
            pl.BlockSpec((H, D), rep),
            pl.BlockSpec((1, D), rep),
            pl.BlockSpec((1, D), rep),
            pl.BlockSpec((1, D), rep),
        ],
        out_specs=pl.BlockSpec((block_m, D), lambda i: (i, 0)),
        out_shape=jax.ShapeDtypeStruct((M, D), jnp.float32),
        interpret=interpret,
    )(xf, W_enc.T, b_enc.reshape(1, K), W1.T, b1.reshape(1, H),
      W2.T, b2.reshape(1, H), W3.T, b3.reshape(1, D),
      ln_w.reshape(1, D), ln_b.reshape(1, D))
    return out.reshape(B, T, D)


# speedup vs baseline: 4.3999x; 1.0025x over previous
import jax
import jax.numpy as jnp
from jax.experimental import pallas as pl
from jax.experimental.pallas import tpu as pltpu


def _gelu_exact(x):
    return 0.5 * x * (1.0 + jax.lax.erf(x * 0.7071067811865476))


def _body(x_ref, wenc_ref, benc_ref, w1_ref, b1_ref, w2_ref, b2_ref,
          w3_ref, b3_ref, lnw_ref, lnb_ref, out_ref):
    x = x_ref[...]
    logits = jnp.dot(x, wenc_ref[...],
                     preferred_element_type=jnp.float32) + benc_ref[...]
    bits = jnp.sign(logits)
    h = jnp.dot(bits, w1_ref[...],
                preferred_element_type=jnp.float32) + b1_ref[...]
    h = _gelu_exact(h)
    h = jnp.dot(h, w2_ref[...],
                preferred_element_type=jnp.float32) + b2_ref[...]
    h = _gelu_exact(h)
    h = jnp.dot(h, w3_ref[...],
                preferred_element_type=jnp.float32) + b3_ref[...]
    mean = jnp.mean(h, axis=-1, keepdims=True)
    var = jnp.mean((h - mean) * (h - mean), axis=-1, keepdims=True)
    out_ref[...] = ((h - mean) * jax.lax.rsqrt(var + 1e-5)
                    * lnw_ref[...] + lnb_ref[...])


def kernel(x, W_enc, b_enc, W1, b1, W2, b2, W3, b3, ln_w, ln_b,
           block_m: int = 512, interpret: bool = False):
    B, T, D = x.shape
    K = W_enc.shape[0]
    H = W1.shape[0]
    M = B * T
    xf = x.reshape(M, D)

    rep = lambda i: (0, 0)
    out = pl.pallas_call(
        _body,
        grid=(M // block_m,),
        in_specs=[
            pl.BlockSpec((block_m, D), lambda i: (i, 0)),
            pl.BlockSpec((D, K), rep),
            pl.BlockSpec((1, K), rep),
            pl.BlockSpec((K, H), rep),
            pl.BlockSpec((1, H), rep),
            pl.BlockSpec((H, H), rep),
            pl.BlockSpec((1, H), rep),
            pl.BlockSpec((H, D), rep),
            pl.BlockSpec((1, D), rep),
            pl.BlockSpec((1, D), rep),
            pl.BlockSpec((1, D), rep),
        ],
        out_specs=pl.BlockSpec((block_m, D), lambda i: (i, 0)),
        out_shape=jax.ShapeDtypeStruct((M, D), jnp.float32),
        compiler_params=pltpu.CompilerParams(
            dimension_semantics=("parallel",)),
        interpret=interpret,
    )(xf, W_enc.T, b_enc.reshape(1, K), W1.T, b1.reshape(1, H),
      W2.T, b2.reshape(1, H), W3.T, b3.reshape(1, D),
      ln_w.reshape(1, D), ln_b.reshape(1, D))
    return out.reshape(B, T, D)


# speedup vs baseline: 4.7279x; 1.0746x over previous
import jax
import jax.numpy as jnp
from jax.experimental import pallas as pl
from jax.experimental.pallas import tpu as pltpu


def _gelu_exact(x):
    return 0.5 * x * (1.0 + jax.lax.erf(x * 0.7071067811865476))


def _body(x_ref, wenc_ref, benc_ref, w1_ref, b1_ref, w2_ref, b2_ref,
          w3_ref, b3_ref, lnw_ref, lnb_ref, out_ref):
    x = x_ref[...]
    logits = jnp.dot(x, wenc_ref[...],
                     preferred_element_type=jnp.float32) + benc_ref[...]
    bits = jnp.sign(logits)
    h = jnp.dot(bits, w1_ref[...],
                preferred_element_type=jnp.float32) + b1_ref[...]
    h = _gelu_exact(h)
    h = jnp.dot(h, w2_ref[...],
                preferred_element_type=jnp.float32) + b2_ref[...]
    h = _gelu_exact(h)
    h = jnp.dot(h, w3_ref[...],
                preferred_element_type=jnp.float32) + b3_ref[...]
    mean = jnp.mean(h, axis=-1, keepdims=True)
    var = jnp.mean((h - mean) * (h - mean), axis=-1, keepdims=True)
    out_ref[...] = ((h - mean) * jax.lax.rsqrt(var + 1e-5)
                    * lnw_ref[...] + lnb_ref[...])


def kernel(x, W_enc, b_enc, W1, b1, W2, b2, W3, b3, ln_w, ln_b,
           block_m: int = 1024, interpret: bool = False):
    B, T, D = x.shape
    K = W_enc.shape[0]
    H = W1.shape[0]
    M = B * T
    xf = x.reshape(M, D)

    rep = lambda i: (0, 0)
    out = pl.pallas_call(
        _body,
        grid=(M // block_m,),
        in_specs=[
            pl.BlockSpec((block_m, D), lambda i: (i, 0)),
            pl.BlockSpec((D, K), rep),
            pl.BlockSpec((1, K), rep),
            pl.BlockSpec((K, H), rep),
            pl.BlockSpec((1, H), rep),
            pl.BlockSpec((H, H), rep),
            pl.BlockSpec((1, H), rep),
            pl.BlockSpec((H, D), rep),
            pl.BlockSpec((1, D), rep),
            pl.BlockSpec((1, D), rep),
            pl.BlockSpec((1, D), rep),
        ],
        out_specs=pl.BlockSpec((block_m, D), lambda i: (i, 0)),
        out_shape=jax.ShapeDtypeStruct((M, D), jnp.float32),
        compiler_params=pltpu.CompilerParams(
            dimension_semantics=("parallel",)),
        interpret=interpret,
    )(xf, W_enc.T, b_enc.reshape(1, K), W1.T, b1.reshape(1, H),
      W2.T, b2.reshape(1, H), W3.T, b3.reshape(1, D),
      ln_w.reshape(1, D), ln_b.reshape(1, D))
    return out.reshape(B, T, D)
